# Initial kernel scaffold; baseline (speedup 1.0000x reference)
import jax
import jax.numpy as jnp
from jax.experimental import pallas as pl


def kernel(x, edge_index, W1, b1, W2, b2):
    raise NotImplementedError("write your pallas kernel here")


# speedup vs baseline: 18.6359x; 18.6359x over previous
import functools

import jax
import jax.numpy as jnp
from jax import lax
from jax.experimental import pallas as pl
from jax.experimental.pallas import tpu as pltpu
from jax.experimental.pallas import tpu_sc as plsc

_NC = 2
_NS = 16
_NW = _NC * _NS
_B = 80
_CH = 16


def _row_split(n, s):
  rb = (n // _NS) // 8 * 8
  last = n - (_NS - 1) * rb
  base = s * rb
  nch = jnp.where(s == _NS - 1, last // _CH, rb // _CH)
  return base, nch

_mesh = functools.partial(
    plsc.VectorSubcoreMesh, core_axis_name="c", subcore_axis_name="s")


def _sc_degree(dst3, n):
  nb = dst3.shape[1]

  @functools.partial(
      pl.kernel,
      out_type=jax.ShapeDtypeStruct((_NC, n, 16), jnp.float32),
      mesh=_mesh(),
      scratch_types=[
          pltpu.VMEM((nb, _B), jnp.int32),
          pltpu.VMEM((_B, 16), jnp.float32),
          pltpu.VMEM((_CH, 16), jnp.float32),
          pltpu.VMEM_SHARED((n, 16), jnp.float32),
      ],
  )
  def deg_kernel(dst_hbm, out_hbm, didx, ones_v, zbuf, accum):
    c = lax.axis_index("c")
    s = lax.axis_index("s")
    wid = s * _NC + c
    pltpu.sync_copy(dst_hbm.at[wid], didx)

    def fill(r, carry):
      zbuf[r] = jnp.zeros((16,), jnp.float32)
      return carry

    lax.fori_loop(0, _CH, fill, 0)

    def fill1(r, carry):
      ones_v[r] = jnp.ones((16,), jnp.float32)
      return carry

    lax.fori_loop(0, _B, fill1, 0)

    base, nch = _row_split(n, s)

    def zero(k, carry):
      pltpu.sync_copy(zbuf, accum.at[pl.ds(base + k * _CH, _CH)])
      return carry

    lax.fori_loop(0, nch, zero, 0)
    plsc.subcore_barrier()

    def body(i, carry):
      pltpu.sync_copy(ones_v, accum.at[didx.at[i]], add=True)
      return carry

    lax.fori_loop(0, nb, body, 0)
    plsc.subcore_barrier()

    def wout(k, carry):
      pltpu.sync_copy(accum.at[pl.ds(base + k * _CH, _CH)],
                      out_hbm.at[c, pl.ds(base + k * _CH, _CH)])
      return carry

    lax.fori_loop(0, nch, wout, 0)

  return deg_kernel(dst3)


def _sc_edge_agg(table, src3, dst3, n, d, stage_table=False):
  nb = src3.shape[1]
  scratch = [
      pltpu.VMEM((nb, _B), jnp.int32),
      pltpu.VMEM((nb, _B), jnp.int32),
      pltpu.VMEM((_B, d), jnp.float32),
      pltpu.VMEM((_CH, d), jnp.float32),
      pltpu.VMEM_SHARED((n, d), jnp.float32),
      pltpu.SemaphoreType.DMA,
  ]
  if stage_table:
    scratch.append(pltpu.VMEM_SHARED((n, d), jnp.float32))

  @functools.partial(
      pl.kernel,
      out_type=jax.ShapeDtypeStruct((_NC, n, d), jnp.float32),
      mesh=_mesh(),
      scratch_types=scratch,
  )
  def agg_kernel(tbl_hbm, src_hbm, dst_hbm, out_hbm,
                 sidx, didx, rows, zbuf, accum, sem, *tbl_sp):
    c = lax.axis_index("c")
    s = lax.axis_index("s")
    wid = s * _NC + c
    pltpu.sync_copy(src_hbm.at[wid], sidx)
    pltpu.sync_copy(dst_hbm.at[wid], didx)

    def fill(r, carry):
      for j in range(d // 16):
        zbuf[r, pl.ds(j * 16, 16)] = jnp.zeros((16,), jnp.float32)
      return carry

    lax.fori_loop(0, _CH, fill, 0)

    base, nch = _row_split(n, s)

    def zero(k, carry):
      pltpu.sync_copy(zbuf, accum.at[pl.ds(base + k * _CH, _CH)])
      return carry

    lax.fori_loop(0, nch, zero, 0)

    if stage_table:
      def stage(k, carry):
        pltpu.sync_copy(tbl_hbm.at[pl.ds(base + k * _CH, _CH)],
                        tbl_sp[0].at[pl.ds(base + k * _CH, _CH)])
        return carry

      lax.fori_loop(0, nch, stage, 0)
      src_tbl = tbl_sp[0]
    else:
      src_tbl = tbl_hbm
    plsc.subcore_barrier()

    def body(i, carry):
      pltpu.async_copy(src_tbl.at[sidx.at[i]], rows, sem).wait()
      pltpu.sync_copy(rows, accum.at[didx.at[i]], add=True)
      return carry

    lax.fori_loop(0, nb, body, 0)
    plsc.subcore_barrier()

    def wout(k, carry):
      pltpu.sync_copy(accum.at[pl.ds(base + k * _CH, _CH)],
                      out_hbm.at[c, pl.ds(base + k * _CH, _CH)])
      return carry

    lax.fori_loop(0, nch, wout, 0)

  return agg_kernel(table, src3, dst3)


_RB = 2000


def _prescale_body(d0, d1, x, xs, dinv16):
  cnt = d0[...][:, 0:1] + d1[...][:, 0:1]
  dinv = lax.rsqrt(cnt + 1.0)
  xs[...] = x[...] * dinv
  dinv16[...] = jnp.broadcast_to(dinv, dinv16.shape)


def _tc_prescale(degp, x):
  n, din = x.shape
  grid = (n // _RB,)
  row = lambda i: (i, 0)
  return pl.pallas_call(
      _prescale_body,
      grid=grid,
      in_specs=[
          pl.BlockSpec((_RB, 16), row),
          pl.BlockSpec((_RB, 16), row),
          pl.BlockSpec((_RB, din), row),
      ],
      out_specs=[
          pl.BlockSpec((_RB, din), row),
          pl.BlockSpec((_RB, 16), row),
      ],
      out_shape=[
          jax.ShapeDtypeStruct((n, din), jnp.float32),
          jax.ShapeDtypeStruct((n, 16), jnp.float32),
      ],
  )(degp[0], degp[1], x)


def _dense_body(g0, g1, x, dinv16, w1, b1, w2p, z_ref, zs_ref):
  dv = dinv16[...][:, 0:1]
  agg = dv * (g0[...] + g1[...]) + (dv * dv) * x[...]
  h = jnp.maximum(
      jnp.dot(agg, w1[...], preferred_element_type=jnp.float32) + b1[...], 0.0)
  z = jnp.dot(h, w2p[...], preferred_element_type=jnp.float32)
  z_ref[...] = z
  zs_ref[...] = jnp.pad(z * dv, ((0, 0), (0, 112)))


def _tc_dense(gp, x, dinv16, w1, b1, w2p):
  n, din = x.shape
  dhid = w1.shape[1]
  grid = (n // _RB,)
  row = lambda i: (i, 0)
  full = lambda i: (0, 0)
  return pl.pallas_call(
      _dense_body,
      grid=grid,
      in_specs=[
          pl.BlockSpec((_RB, din), row),
          pl.BlockSpec((_RB, din), row),
          pl.BlockSpec((_RB, din), row),
          pl.BlockSpec((_RB, 16), row),
          pl.BlockSpec((din, dhid), full),
          pl.BlockSpec((1, dhid), full),
          pl.BlockSpec((dhid, 16), full),
      ],
      out_specs=[
          pl.BlockSpec((_RB, 16), row),
          pl.BlockSpec((_RB, 128), row),
      ],
      out_shape=[
          jax.ShapeDtypeStruct((n, 16), jnp.float32),
          jax.ShapeDtypeStruct((n, 128), jnp.float32),
      ],
  )(gp[0], gp[1], x, dinv16, w1, b1, w2p)


def _out_body(ncls, g0, g1, z, dinv16, b2p, o_ref):
  dv = dinv16[...][:, 0:1]
  logits = dv * (g0[...][:, :16] + g1[...][:, :16]) + (dv * dv) * z[...] + b2p[...]
  mask = lax.broadcasted_iota(jnp.int32, logits.shape, 1) < ncls
  lm = jnp.where(mask, logits, -1e30)
  m = jnp.max(lm, axis=1, keepdims=True)
  ls = lm - m
  e = jnp.where(mask, jnp.exp(ls), 0.0)
  o_ref[...] = ls - jnp.log(jnp.sum(e, axis=1, keepdims=True))


def _tc_out(g2p, z, dinv16, b2p, ncls):
  n = z.shape[0]
  grid = (n // _RB,)
  row = lambda i: (i, 0)
  full = lambda i: (0, 0)
  return pl.pallas_call(
      functools.partial(_out_body, ncls),
      grid=grid,
      in_specs=[
          pl.BlockSpec((_RB, 128), row),
          pl.BlockSpec((_RB, 128), row),
          pl.BlockSpec((_RB, 16), row),
          pl.BlockSpec((_RB, 16), row),
          pl.BlockSpec((1, 16), full),
      ],
      out_specs=pl.BlockSpec((_RB, 16), row),
      out_shape=jax.ShapeDtypeStruct((n, 16), jnp.float32),
  )(g2p[0], g2p[1], z, dinv16, b2p)


def kernel(x, edge_index, W1, b1, W2, b2):
  n, din = x.shape
  dhid = W1.shape[1]
  ncls = W2.shape[1]
  e = edge_index.shape[1]
  assert e % (_NW * _B) == 0 and n % _CH == 0 and din % 16 == 0

  src3 = edge_index[0].reshape(_NW, -1, _B)
  dst3 = edge_index[1].reshape(_NW, -1, _B)

  degp = _sc_degree(dst3, n)
  xs, dinv16 = _tc_prescale(degp, x)
  gp = _sc_edge_agg(xs, src3, dst3, n, din)

  w2p = jnp.pad(W2, ((0, 0), (0, 16 - ncls)))
  b2p = jnp.pad(b2, (0, 16 - ncls)).reshape(1, 16)
  z, zs = _tc_dense(gp, x, dinv16, W1, b1.reshape(1, dhid), w2p)

  g2p = _sc_edge_agg(zs, src3, dst3, n, 128)
  out16 = _tc_out(g2p, z, dinv16, b2p, ncls)
  return out16[:, :ncls]


# speedup vs baseline: 28.5704x; 1.5331x over previous
import functools

import jax
import jax.numpy as jnp
from jax import lax
from jax.experimental import pallas as pl
from jax.experimental.pallas import tpu as pltpu
from jax.experimental.pallas import tpu_sc as plsc

_NC = 2
_NS = 16
_NW = _NC * _NS
_B = 125
_CH = 16


def _row_split(n, s):
  rb = (n // _NS) // 8 * 8
  last = n - (_NS - 1) * rb
  base = s * rb
  nch = jnp.where(s == _NS - 1, last // _CH, rb // _CH)
  return base, nch

_mesh = functools.partial(
    plsc.VectorSubcoreMesh, core_axis_name="c", subcore_axis_name="s")


def _sc_degree(dst3, n):
  nb = dst3.shape[1]

  @functools.partial(
      pl.kernel,
      out_type=jax.ShapeDtypeStruct((_NC, n, 16), jnp.float32),
      mesh=_mesh(),
      scratch_types=[
          pltpu.VMEM((nb, _B), jnp.int32),
          pltpu.VMEM((_B, 16), jnp.float32),
          pltpu.VMEM((_CH, 16), jnp.float32),
          pltpu.VMEM_SHARED((n, 16), jnp.float32),
          pltpu.SemaphoreType.DMA,
      ],
  )
  def deg_kernel(dst_hbm, out_hbm, didx, ones_v, zbuf, accum, sem):
    c = lax.axis_index("c")
    s = lax.axis_index("s")
    wid = s * _NC + c
    pltpu.sync_copy(dst_hbm.at[wid], didx)

    def fill(r, carry):
      zbuf[r] = jnp.zeros((16,), jnp.float32)
      return carry

    lax.fori_loop(0, _CH, fill, 0)

    def fill1(r, carry):
      ones_v[r] = jnp.ones((16,), jnp.float32)
      return carry

    lax.fori_loop(0, _B, fill1, 0)

    base, nch = _row_split(n, s)

    def zero(k, carry):
      pltpu.sync_copy(zbuf, accum.at[pl.ds(base + k * _CH, _CH)])
      return carry

    lax.fori_loop(0, nch, zero, 0)
    plsc.subcore_barrier()

    grp = 10

    def body(g, carry):
      for j in range(grp):
        pltpu.async_copy(ones_v, accum.at[didx.at[g * grp + j]], sem, add=True)
      for j in range(grp):
        pltpu.make_async_copy(ones_v, accum.at[didx.at[g * grp + j]], sem).wait()
      return carry

    lax.fori_loop(0, nb // grp, body, 0)
    plsc.subcore_barrier()

    def wout(k, carry):
      pltpu.sync_copy(accum.at[pl.ds(base + k * _CH, _CH)],
                      out_hbm.at[c, pl.ds(base + k * _CH, _CH)])
      return carry

    lax.fori_loop(0, nch, wout, 0)

  return deg_kernel(dst3)


def _sc_edge_agg(table, idx4, n, d):
  nb = idx4.shape[1]
  assert nb % 2 == 0 and nb >= 4

  @functools.partial(
      pl.kernel,
      out_type=jax.ShapeDtypeStruct((_NC, n, d), jnp.float32),
      mesh=_mesh(),
      scratch_types=[
          pltpu.VMEM((4, 2, _B), jnp.int32),
          pltpu.VMEM((2, _B, d), jnp.float32),
          pltpu.VMEM((_CH, d), jnp.float32),
          pltpu.VMEM_SHARED((n, d), jnp.float32),
          pltpu.SemaphoreType.DMA,
          pltpu.SemaphoreType.DMA,
          pltpu.SemaphoreType.DMA,
          pltpu.SemaphoreType.DMA,
      ],
  )
  def agg_kernel(tbl_hbm, idx_hbm, out_hbm,
                 ring, rows, zbuf, accum, sem0, sem1, semi0, semi1):
    c = lax.axis_index("c")
    s = lax.axis_index("s")
    wid = s * _NC + c
    for i in range(4):
      pltpu.sync_copy(idx_hbm.at[wid, i], ring.at[i])

    def fill(r, carry):
      for j in range(d // 16):
        zbuf[r, pl.ds(j * 16, 16)] = jnp.zeros((16,), jnp.float32)
      return carry

    lax.fori_loop(0, _CH, fill, 0)

    base, nch = _row_split(n, s)

    def zero(k, carry):
      pltpu.sync_copy(zbuf, accum.at[pl.ds(base + k * _CH, _CH)])
      return carry

    lax.fori_loop(0, nch, zero, 0)
    plsc.subcore_barrier()

    sems = (sem0, sem1)
    isems = (semi0, semi1)
    pltpu.async_copy(tbl_hbm.at[ring.at[0, 0]], rows.at[0], sem0)
    pltpu.async_copy(tbl_hbm.at[ring.at[1, 0]], rows.at[1], sem1)

    def body(k, carry):
      for b in range(2):
        i = 2 * k + b
        pltpu.make_async_copy(tbl_hbm.at[ring.at[i % 4, 0]], rows.at[b],
                              sems[b]).wait()
        pltpu.sync_copy(rows.at[b], accum.at[ring.at[i % 4, 1]], add=True)

        @pl.when(i + 4 < nb)
        def _():
          pltpu.async_copy(idx_hbm.at[wid, i + 4], ring.at[i % 4], isems[b])

        @pl.when(i + 2 < nb)
        def _():
          @pl.when(i >= 2)
          def _():
            pltpu.make_async_copy(idx_hbm.at[wid, i + 2],
                                  ring.at[(i + 2) % 4], isems[b]).wait()
          pltpu.async_copy(tbl_hbm.at[ring.at[(i + 2) % 4, 0]], rows.at[b],
                           sems[b])
      return carry

    lax.fori_loop(0, nb // 2, body, 0)
    plsc.subcore_barrier()

    def wout(k, carry):
      pltpu.sync_copy(accum.at[pl.ds(base + k * _CH, _CH)],
                      out_hbm.at[c, pl.ds(base + k * _CH, _CH)])
      return carry

    lax.fori_loop(0, nch, wout, 0)

  return agg_kernel(table, idx4)


_RB = 2000


def _prescale_body(d0, d1, x, xs, dinv16):
  cnt = d0[...][:, 0:1] + d1[...][:, 0:1]
  dinv = lax.rsqrt(cnt + 1.0)
  xs[...] = x[...] * dinv
  dinv16[...] = jnp.broadcast_to(dinv, dinv16.shape)


def _tc_prescale(degp, x):
  n, din = x.shape
  grid = (n // _RB,)
  row = lambda i: (i, 0)
  return pl.pallas_call(
      _prescale_body,
      grid=grid,
      in_specs=[
          pl.BlockSpec((_RB, 16), row),
          pl.BlockSpec((_RB, 16), row),
          pl.BlockSpec((_RB, din), row),
      ],
      out_specs=[
          pl.BlockSpec((_RB, din), row),
          pl.BlockSpec((_RB, 16), row),
      ],
      out_shape=[
          jax.ShapeDtypeStruct((n, din), jnp.float32),
          jax.ShapeDtypeStruct((n, 16), jnp.float32),
      ],
  )(degp[0], degp[1], x)


def _dense_body(g0, g1, x, dinv16, w1, b1, w2p, z_ref, zs_ref):
  dv = dinv16[...][:, 0:1]
  agg = dv * (g0[...] + g1[...]) + (dv * dv) * x[...]
  h = jnp.maximum(
      jnp.dot(agg, w1[...], preferred_element_type=jnp.float32) + b1[...], 0.0)
  z = jnp.dot(h, w2p[...], preferred_element_type=jnp.float32)
  z_ref[...] = z
  zs_ref[...] = jnp.pad(z * dv, ((0, 0), (0, 112)))


def _tc_dense(gp, x, dinv16, w1, b1, w2p):
  n, din = x.shape
  dhid = w1.shape[1]
  grid = (n // _RB,)
  row = lambda i: (i, 0)
  full = lambda i: (0, 0)
  return pl.pallas_call(
      _dense_body,
      grid=grid,
      in_specs=[
          pl.BlockSpec((_RB, din), row),
          pl.BlockSpec((_RB, din), row),
          pl.BlockSpec((_RB, din), row),
          pl.BlockSpec((_RB, 16), row),
          pl.BlockSpec((din, dhid), full),
          pl.BlockSpec((1, dhid), full),
          pl.BlockSpec((dhid, 16), full),
      ],
      out_specs=[
          pl.BlockSpec((_RB, 16), row),
          pl.BlockSpec((_RB, 128), row),
      ],
      out_shape=[
          jax.ShapeDtypeStruct((n, 16), jnp.float32),
          jax.ShapeDtypeStruct((n, 128), jnp.float32),
      ],
  )(gp[0], gp[1], x, dinv16, w1, b1, w2p)


def _out_body(ncls, g0, g1, z, dinv16, b2p, o_ref):
  dv = dinv16[...][:, 0:1]
  logits = dv * (g0[...][:, :16] + g1[...][:, :16]) + (dv * dv) * z[...] + b2p[...]
  mask = lax.broadcasted_iota(jnp.int32, logits.shape, 1) < ncls
  lm = jnp.where(mask, logits, -1e30)
  m = jnp.max(lm, axis=1, keepdims=True)
  ls = lm - m
  e = jnp.where(mask, jnp.exp(ls), 0.0)
  o_ref[...] = ls - jnp.log(jnp.sum(e, axis=1, keepdims=True))


def _tc_out(g2p, z, dinv16, b2p, ncls):
  n = z.shape[0]
  grid = (n // _RB,)
  row = lambda i: (i, 0)
  full = lambda i: (0, 0)
  return pl.pallas_call(
      functools.partial(_out_body, ncls),
      grid=grid,
      in_specs=[
          pl.BlockSpec((_RB, 128), row),
          pl.BlockSpec((_RB, 128), row),
          pl.BlockSpec((_RB, 16), row),
          pl.BlockSpec((_RB, 16), row),
          pl.BlockSpec((1, 16), full),
      ],
      out_specs=pl.BlockSpec((_RB, 16), row),
      out_shape=jax.ShapeDtypeStruct((n, 16), jnp.float32),
  )(g2p[0], g2p[1], z, dinv16, b2p)


def kernel(x, edge_index, W1, b1, W2, b2):
  n, din = x.shape
  dhid = W1.shape[1]
  ncls = W2.shape[1]
  e = edge_index.shape[1]
  assert e % (_NW * _B) == 0 and n % _CH == 0 and din % 16 == 0

  src3 = edge_index[0].reshape(_NW, -1, _B)
  dst3 = edge_index[1].reshape(_NW, -1, _B)
  idx4 = jnp.stack([src3, dst3], axis=2)

  degp = _sc_degree(dst3, n)
  xs, dinv16 = _tc_prescale(degp, x)
  gp = _sc_edge_agg(xs, idx4, n, din)

  w2p = jnp.pad(W2, ((0, 0), (0, 16 - ncls)))
  b2p = jnp.pad(b2, (0, 16 - ncls)).reshape(1, 16)
  z, zs = _tc_dense(gp, x, dinv16, W1, b1.reshape(1, dhid), w2p)

  g2p = _sc_edge_agg(zs, idx4, n, 128)
  out16 = _tc_out(g2p, z, dinv16, b2p, ncls)
  return out16[:, :ncls]


# speedup vs baseline: 28.6299x; 1.0021x over previous
import functools

import jax
import jax.numpy as jnp
from jax import lax
from jax.experimental import pallas as pl
from jax.experimental.pallas import tpu as pltpu
from jax.experimental.pallas import tpu_sc as plsc

_NC = 2
_NS = 16
_NW = _NC * _NS
_B = 125
_CH = 16


def _row_split(n, s):
  rb = (n // _NS) // 8 * 8
  last = n - (_NS - 1) * rb
  base = s * rb
  nch = jnp.where(s == _NS - 1, last // _CH, rb // _CH)
  return base, nch

_mesh = functools.partial(
    plsc.VectorSubcoreMesh, core_axis_name="c", subcore_axis_name="s")


def _sc_degree(dst3, n):
  nb = dst3.shape[1]

  @functools.partial(
      pl.kernel,
      out_type=jax.ShapeDtypeStruct((_NC, n, 16), jnp.float32),
      mesh=_mesh(),
      scratch_types=[
          pltpu.VMEM((nb, _B), jnp.int32),
          pltpu.VMEM((_B, 16), jnp.float32),
          pltpu.VMEM((_CH, 16), jnp.float32),
          pltpu.VMEM_SHARED((n, 16), jnp.float32),
          pltpu.SemaphoreType.DMA,
      ],
  )
  def deg_kernel(dst_hbm, out_hbm, didx, ones_v, zbuf, accum, sem):
    c = lax.axis_index("c")
    s = lax.axis_index("s")
    wid = s * _NC + c
    pltpu.sync_copy(dst_hbm.at[wid], didx)

    def fill(r, carry):
      zbuf[r] = jnp.zeros((16,), jnp.float32)
      return carry

    lax.fori_loop(0, _CH, fill, 0)

    def fill1(r, carry):
      ones_v[r] = jnp.ones((16,), jnp.float32)
      return carry

    lax.fori_loop(0, _B, fill1, 0)

    base, nch = _row_split(n, s)

    def zero(k, carry):
      pltpu.sync_copy(zbuf, accum.at[pl.ds(base + k * _CH, _CH)])
      return carry

    lax.fori_loop(0, nch, zero, 0)
    plsc.subcore_barrier()

    grp = 10

    def body(g, carry):
      for j in range(grp):
        pltpu.async_copy(ones_v, accum.at[didx.at[g * grp + j]], sem, add=True)
      for j in range(grp):
        pltpu.make_async_copy(ones_v, accum.at[didx.at[g * grp + j]], sem).wait()
      return carry

    lax.fori_loop(0, nb // grp, body, 0)
    plsc.subcore_barrier()

    def wout(k, carry):
      pltpu.sync_copy(accum.at[pl.ds(base + k * _CH, _CH)],
                      out_hbm.at[c, pl.ds(base + k * _CH, _CH)])
      return carry

    lax.fori_loop(0, nch, wout, 0)

  return deg_kernel(dst3)


def _sc_edge_agg(table, idx4, n, d, dtype=jnp.float32):
  nb = idx4.shape[1]
  assert nb % 2 == 0 and nb >= 4
  assert table.dtype == dtype

  @functools.partial(
      pl.kernel,
      out_type=jax.ShapeDtypeStruct((_NC, n, d), dtype),
      mesh=_mesh(),
      scratch_types=[
          pltpu.VMEM((4, 2, _B), jnp.int32),
          pltpu.VMEM((2, _B, d), dtype),
          pltpu.VMEM((_CH, d), dtype),
          pltpu.VMEM_SHARED((n, d), dtype),
          pltpu.SemaphoreType.DMA,
          pltpu.SemaphoreType.DMA,
          pltpu.SemaphoreType.DMA,
          pltpu.SemaphoreType.DMA,
      ],
  )
  def agg_kernel(tbl_hbm, idx_hbm, out_hbm,
                 ring, rows, zbuf, accum, sem0, sem1, semi0, semi1):
    c = lax.axis_index("c")
    s = lax.axis_index("s")
    wid = s * _NC + c
    for i in range(4):
      pltpu.sync_copy(idx_hbm.at[wid, i], ring.at[i])

    def fill(r, carry):
      for j in range(d // 16):
        zbuf[r, pl.ds(j * 16, 16)] = jnp.zeros((16,), dtype)
      return carry

    lax.fori_loop(0, _CH, fill, 0)

    base, nch = _row_split(n, s)

    def zero(k, carry):
      pltpu.sync_copy(zbuf, accum.at[pl.ds(base + k * _CH, _CH)])
      return carry

    lax.fori_loop(0, nch, zero, 0)
    plsc.subcore_barrier()

    sems = (sem0, sem1)
    isems = (semi0, semi1)
    pltpu.async_copy(tbl_hbm.at[ring.at[0, 0]], rows.at[0], sem0)
    pltpu.async_copy(tbl_hbm.at[ring.at[1, 0]], rows.at[1], sem1)

    def body(k, carry):
      for b in range(2):
        i = 2 * k + b
        pltpu.make_async_copy(tbl_hbm.at[ring.at[i % 4, 0]], rows.at[b],
                              sems[b]).wait()
        pltpu.sync_copy(rows.at[b], accum.at[ring.at[i % 4, 1]], add=True)

        @pl.when(i + 4 < nb)
        def _():
          pltpu.async_copy(idx_hbm.at[wid, i + 4], ring.at[i % 4], isems[b])

        @pl.when(i + 2 < nb)
        def _():
          @pl.when(i >= 2)
          def _():
            pltpu.make_async_copy(idx_hbm.at[wid, i + 2],
                                  ring.at[(i + 2) % 4], isems[b]).wait()
          pltpu.async_copy(tbl_hbm.at[ring.at[(i + 2) % 4, 0]], rows.at[b],
                           sems[b])
      return carry

    lax.fori_loop(0, nb // 2, body, 0)
    plsc.subcore_barrier()

    def wout(k, carry):
      pltpu.sync_copy(accum.at[pl.ds(base + k * _CH, _CH)],
                      out_hbm.at[c, pl.ds(base + k * _CH, _CH)])
      return carry

    lax.fori_loop(0, nch, wout, 0)

  return agg_kernel(table, idx4)


_RB = 2000


def _prescale_body(d0, d1, x, xs, dinv16):
  cnt = d0[...][:, 0:1] + d1[...][:, 0:1]
  dinv = lax.rsqrt(cnt + 1.0)
  xs[...] = x[...] * dinv
  dinv16[...] = jnp.broadcast_to(dinv, dinv16.shape)


def _tc_prescale(degp, x):
  n, din = x.shape
  grid = (n // _RB,)
  row = lambda i: (i, 0)
  return pl.pallas_call(
      _prescale_body,
      grid=grid,
      in_specs=[
          pl.BlockSpec((_RB, 16), row),
          pl.BlockSpec((_RB, 16), row),
          pl.BlockSpec((_RB, din), row),
      ],
      out_specs=[
          pl.BlockSpec((_RB, din), row),
          pl.BlockSpec((_RB, 16), row),
      ],
      out_shape=[
          jax.ShapeDtypeStruct((n, din), jnp.float32),
          jax.ShapeDtypeStruct((n, 16), jnp.float32),
      ],
  )(degp[0], degp[1], x)


def _dense_body(g0, g1, x, dinv16, w1, b1, w2p, z_ref, zs_ref):
  dv = dinv16[...][:, 0:1]
  agg = dv * (g0[...] + g1[...]) + (dv * dv) * x[...]
  h = jnp.maximum(
      jnp.dot(agg, w1[...], preferred_element_type=jnp.float32) + b1[...], 0.0)
  z = jnp.dot(h, w2p[...], preferred_element_type=jnp.float32)
  z_ref[...] = z
  zs_ref[...] = jnp.pad(z * dv, ((0, 0), (0, 112)))


def _tc_dense(gp, x, dinv16, w1, b1, w2p):
  n, din = x.shape
  dhid = w1.shape[1]
  grid = (n // _RB,)
  row = lambda i: (i, 0)
  full = lambda i: (0, 0)
  return pl.pallas_call(
      _dense_body,
      grid=grid,
      in_specs=[
          pl.BlockSpec((_RB, din), row),
          pl.BlockSpec((_RB, din), row),
          pl.BlockSpec((_RB, din), row),
          pl.BlockSpec((_RB, 16), row),
          pl.BlockSpec((din, dhid), full),
          pl.BlockSpec((1, dhid), full),
          pl.BlockSpec((dhid, 16), full),
      ],
      out_specs=[
          pl.BlockSpec((_RB, 16), row),
          pl.BlockSpec((_RB, 128), row),
      ],
      out_shape=[
          jax.ShapeDtypeStruct((n, 16), jnp.float32),
          jax.ShapeDtypeStruct((n, 128), jnp.float32),
      ],
  )(gp[0], gp[1], x, dinv16, w1, b1, w2p)


def _out_body(ncls, g0, g1, z, dinv16, b2p, o_ref):
  dv = dinv16[...][:, 0:1]
  logits = dv * (g0[...][:, :16] + g1[...][:, :16]) + (dv * dv) * z[...] + b2p[...]
  mask = lax.broadcasted_iota(jnp.int32, logits.shape, 1) < ncls
  lm = jnp.where(mask, logits, -1e30)
  m = jnp.max(lm, axis=1, keepdims=True)
  ls = lm - m
  e = jnp.where(mask, jnp.exp(ls), 0.0)
  o_ref[...] = ls - jnp.log(jnp.sum(e, axis=1, keepdims=True))


def _tc_out(g2p, z, dinv16, b2p, ncls):
  n = z.shape[0]
  grid = (n // _RB,)
  row = lambda i: (i, 0)
  full = lambda i: (0, 0)
  return pl.pallas_call(
      functools.partial(_out_body, ncls),
      grid=grid,
      in_specs=[
          pl.BlockSpec((_RB, 128), row),
          pl.BlockSpec((_RB, 128), row),
          pl.BlockSpec((_RB, 16), row),
          pl.BlockSpec((_RB, 16), row),
          pl.BlockSpec((1, 16), full),
      ],
      out_specs=pl.BlockSpec((_RB, 16), row),
      out_shape=jax.ShapeDtypeStruct((n, 16), jnp.float32),
  )(g2p[0], g2p[1], z, dinv16, b2p)


def kernel(x, edge_index, W1, b1, W2, b2):
  n, din = x.shape
  dhid = W1.shape[1]
  ncls = W2.shape[1]
  e = edge_index.shape[1]
  assert e % (_NW * _B) == 0 and n % _CH == 0 and din % 16 == 0

  src3 = edge_index[0].reshape(_NW, -1, _B)
  dst3 = edge_index[1].reshape(_NW, -1, _B)
  idx4 = jnp.stack([src3, dst3], axis=2)

  degp = _sc_degree(dst3, n)
  xs, dinv16 = _tc_prescale(degp, x)
  gp = _sc_edge_agg(xs, idx4, n, din)

  w2p = jnp.pad(W2, ((0, 0), (0, 16 - ncls)))
  b2p = jnp.pad(b2, (0, 16 - ncls)).reshape(1, 16)
  z, zs = _tc_dense(gp, x, dinv16, W1, b1.reshape(1, dhid), w2p)

  g2p = _sc_edge_agg(zs, idx4, n, 128)
  out16 = _tc_out(g2p, z, dinv16, b2p, ncls)
  return out16[:, :ncls]
